# Initial kernel scaffold; baseline (speedup 1.0000x reference)
import jax
import jax.numpy as jnp
from jax.experimental import pallas as pl


def kernel(locations, confidences, prior_boxes):
    raise NotImplementedError("write your pallas kernel here")


# speedup vs baseline: 1.7569x; 1.7569x over previous
import jax
import jax.numpy as jnp
from jax import lax
from jax.experimental import pallas as pl
from jax.experimental.pallas import tpu as pltpu
from jax.experimental.pallas import tpu_sc as plsc

C = 21
FG = C - 1
N = 20000
TOP_K = 200
NMS_T = 0.45
CONF_T = 0.01
VAR0, VAR1 = 0.1, 0.2
NEG = -1e9
L = 16
CHUNKS = N // L
NC = 2
ROWW = 16


def _prep_body(conf_ref, loc_ref, pri_ref, scores_ref, boxes_ref):
    conf = conf_ref[...]
    m = jnp.max(conf, axis=0, keepdims=True)
    e = jnp.exp(conf - m)
    p = e / jnp.sum(e, axis=0, keepdims=True)
    fg = p[1:, :]
    scores_ref[...] = jnp.where(fg > CONF_T, fg, NEG)

    lxy = loc_ref[0:2, :]
    lwh = loc_ref[2:4, :]
    pxy = pri_ref[0:2, :]
    pwh = pri_ref[2:4, :]
    xy = pxy + lxy * VAR0 * pwh
    wh = pwh * jnp.exp(lwh * VAR1)
    tl = xy - wh / 2.0
    br = tl + wh
    area = (br[0:1] - tl[0:1]) * (br[1:2] - tl[1:2])
    boxes_ref[...] = jnp.concatenate([tl, br, area], axis=0)


def _prep(conf_t, loc_t, pri_t):
    return pl.pallas_call(
        _prep_body,
        out_shape=[
            jax.ShapeDtypeStruct((FG, N), jnp.float32),
            jax.ShapeDtypeStruct((5, N), jnp.float32),
        ],
    )(conf_t, loc_t, pri_t)


def _nms_body(scores_hbm, boxes_hbm, out_hbm,
              s_v, x1_v, y1_v, x2_v, y2_v, ar_v, row_v):
    cid = lax.axis_index("c")
    sid = lax.axis_index("s")
    wid = sid * NC + cid

    @pl.when(wid < FG)
    def _work():
        pltpu.sync_copy(scores_hbm.at[pl.ds(wid * N, N)], s_v)
        pltpu.sync_copy(boxes_hbm.at[pl.ds(0 * N, N)], x1_v)
        pltpu.sync_copy(boxes_hbm.at[pl.ds(1 * N, N)], y1_v)
        pltpu.sync_copy(boxes_hbm.at[pl.ds(2 * N, N)], x2_v)
        pltpu.sync_copy(boxes_hbm.at[pl.ds(3 * N, N)], y2_v)
        pltpu.sync_copy(boxes_hbm.at[pl.ds(4 * N, N)], ar_v)

        def sweep(bx1, by1, bx2, by2, bar, bidx):
            def chunk(j, carry):
                mx, mi, bx1, by1, bx2, by2, bar, bidx = carry
                lane = lax.iota(jnp.int32, L)
                base = j * L
                s = s_v[pl.ds(base, L)]
                x1 = x1_v[pl.ds(base, L)]
                y1 = y1_v[pl.ds(base, L)]
                x2 = x2_v[pl.ds(base, L)]
                y2 = y2_v[pl.ds(base, L)]
                ar = ar_v[pl.ds(base, L)]
                ix1 = jnp.maximum(x1, bx1)
                iy1 = jnp.maximum(y1, by1)
                ix2 = jnp.minimum(x2, bx2)
                iy2 = jnp.minimum(y2, by2)
                iw = jnp.maximum(ix2 - ix1, 0.0)
                ih = jnp.maximum(iy2 - iy1, 0.0)
                inter = iw * ih
                denom = ((bar + ar) - inter) + 1e-9
                iou = inter / denom
                gi = base + lane
                kill = (iou > NMS_T) | (gi == bidx)
                s2 = jnp.where(kill, jnp.full((L,), NEG, jnp.float32), s)
                s_v[pl.ds(base, L)] = s2
                upd = s2 > mx
                return (jnp.where(upd, s2, mx), jnp.where(upd, gi, mi),
                        bx1, by1, bx2, by2, bar, bidx)

            m0 = jnp.full((L,), -3.0e38, jnp.float32)
            i0 = jnp.zeros((L,), jnp.int32)
            out = lax.fori_loop(0, CHUNKS, chunk,
                                (m0, i0, bx1, by1, bx2, by2, bar, bidx))
            return out[0], out[1]

        far = jnp.full((L,), 1e30, jnp.float32)
        m, mi = sweep(far, far, far, far, jnp.zeros((L,), jnp.float32),
                      jnp.full((L,), -1, jnp.int32))

        def step(k, carry):
            mx, mi = carry
            lane = lax.iota(jnp.int32, L)
            gm = jnp.max(mx)
            gmv = jnp.broadcast_to(gm, (L,))
            cand = jnp.where(mx == gmv, mi, jnp.int32(2**30))
            idx = jnp.min(cand)
            idxv = jnp.broadcast_to(idx, (L,))
            bx1 = plsc.load_gather(x1_v, [idxv])
            by1 = plsc.load_gather(y1_v, [idxv])
            bx2 = plsc.load_gather(x2_v, [idxv])
            by2 = plsc.load_gather(y2_v, [idxv])
            bar = plsc.load_gather(ar_v, [idxv])
            validv = jnp.where(gmv > CONF_T,
                               jnp.ones((L,), jnp.float32),
                               jnp.zeros((L,), jnp.float32))
            row = ((lane == 0).astype(jnp.float32) * gmv
                   + (lane == 1).astype(jnp.float32) * bx1
                   + (lane == 2).astype(jnp.float32) * by1
                   + (lane == 3).astype(jnp.float32) * bx2
                   + (lane == 4).astype(jnp.float32) * by2) * validv
            row_v[pl.ds(k * ROWW, ROWW)] = row
            return sweep(bx1, by1, bx2, by2, bar, idxv)

        lax.fori_loop(0, TOP_K, step, (m, mi))
        pltpu.sync_copy(row_v, out_hbm.at[pl.ds(wid * TOP_K * ROWW, TOP_K * ROWW)])


def _nms(scores, boxes):
    mesh = plsc.VectorSubcoreMesh(core_axis_name="c", subcore_axis_name="s")
    return pl.kernel(
        _nms_body,
        out_type=jax.ShapeDtypeStruct((FG * TOP_K * ROWW,), jnp.float32),
        mesh=mesh,
        compiler_params=pltpu.CompilerParams(needs_layout_passes=False),
        scratch_types=[
            pltpu.VMEM((N,), jnp.float32),
            pltpu.VMEM((N,), jnp.float32),
            pltpu.VMEM((N,), jnp.float32),
            pltpu.VMEM((N,), jnp.float32),
            pltpu.VMEM((N,), jnp.float32),
            pltpu.VMEM((N,), jnp.float32),
            pltpu.VMEM((TOP_K * ROWW,), jnp.float32),
        ],
    )(scores, boxes)


def kernel(locations, confidences, prior_boxes):
    conf_t = confidences[0].T
    loc_t = locations[0].T
    pri_t = prior_boxes.T
    scores, boxes = _prep(conf_t, loc_t, pri_t)
    rows = _nms(scores.reshape(-1), boxes.reshape(-1))
    fg = rows.reshape(FG, TOP_K, ROWW)[:, :, :5]
    bg = jnp.zeros((1, TOP_K, 5), jnp.float32)
    return jnp.concatenate([bg, fg], axis=0)[None]


# speedup vs baseline: 7.2785x; 4.1428x over previous
import jax
import jax.numpy as jnp
from jax import lax
from jax.experimental import pallas as pl
from jax.experimental.pallas import tpu as pltpu
from jax.experimental.pallas import tpu_sc as plsc

C = 21
FG = C - 1
N = 20000
TOP_K = 200
NMS_T = 0.45
CONF_T = 0.01
VAR0, VAR1 = 0.1, 0.2
NEG = -1e9
FAR = 1e30
L = 16
U = 4
NP = 20480
ITERS = NP // (U * L)
NC = 2
ROWW = 16


def _prep_body(conf_ref, loc_ref, pri_ref, scores_ref, boxes_ref):
    conf = conf_ref[...]
    m = jnp.max(conf, axis=0, keepdims=True)
    e = jnp.exp(conf - m)
    p = e / jnp.sum(e, axis=0, keepdims=True)
    fg = p[1:, :]
    spad = jnp.full((FG, NP - N), NEG, jnp.float32)
    scores_ref[...] = jnp.concatenate(
        [jnp.where(fg > CONF_T, fg, NEG), spad], axis=1)

    lxy = loc_ref[0:2, :]
    lwh = loc_ref[2:4, :]
    pxy = pri_ref[0:2, :]
    pwh = pri_ref[2:4, :]
    xy = pxy + lxy * VAR0 * pwh
    wh = pwh * jnp.exp(lwh * VAR1)
    tl = xy - wh / 2.0
    br = tl + wh
    bpad = jnp.full((4, NP - N), FAR, jnp.float32)
    boxes_ref[...] = jnp.concatenate(
        [jnp.concatenate([tl, br], axis=0), bpad], axis=1)


def _prep(conf_t, loc_t, pri_t):
    return pl.pallas_call(
        _prep_body,
        out_shape=[
            jax.ShapeDtypeStruct((FG, NP), jnp.float32),
            jax.ShapeDtypeStruct((4, NP), jnp.float32),
        ],
    )(conf_t, loc_t, pri_t)


def _nms_body(scores_hbm, boxes_hbm, out_hbm,
              s_v, x1_v, y1_v, x2_v, y2_v, row_v):
    cid = lax.axis_index("c")
    sid = lax.axis_index("s")
    wid = sid * NC + cid

    @pl.when(wid < FG)
    def _work():
        pltpu.sync_copy(scores_hbm.at[pl.ds(wid * NP, NP)], s_v)
        pltpu.sync_copy(boxes_hbm.at[pl.ds(0 * NP, NP)], x1_v)
        pltpu.sync_copy(boxes_hbm.at[pl.ds(1 * NP, NP)], y1_v)
        pltpu.sync_copy(boxes_hbm.at[pl.ds(2 * NP, NP)], x2_v)
        pltpu.sync_copy(boxes_hbm.at[pl.ds(3 * NP, NP)], y2_v)

        def sweep(bx1, by1, bx2, by2, bar):
            def chunk(j, carry):
                accs = carry[:2 * U]
                bx1, by1, bx2, by2, bar = carry[2 * U:]
                out = []
                for u in range(U):
                    mx, mi = accs[2 * u], accs[2 * u + 1]
                    lane = lax.iota(jnp.int32, L)
                    base = j * (U * L) + u * L
                    s = s_v[pl.ds(base, L)]
                    x1 = x1_v[pl.ds(base, L)]
                    y1 = y1_v[pl.ds(base, L)]
                    x2 = x2_v[pl.ds(base, L)]
                    y2 = y2_v[pl.ds(base, L)]
                    ar = (x2 - x1) * (y2 - y1)
                    ix1 = jnp.maximum(x1, bx1)
                    iy1 = jnp.maximum(y1, by1)
                    ix2 = jnp.minimum(x2, bx2)
                    iy2 = jnp.minimum(y2, by2)
                    iw = jnp.maximum(ix2 - ix1, 0.0)
                    ih = jnp.maximum(iy2 - iy1, 0.0)
                    inter = iw * ih
                    denom = ((bar + ar) - inter) + 1e-9
                    iou = inter / denom
                    kill = iou > NMS_T
                    s2 = jnp.where(kill, jnp.full((L,), NEG, jnp.float32), s)
                    s_v[pl.ds(base, L)] = s2
                    upd = s2 > mx
                    out.append(jnp.where(upd, s2, mx))
                    out.append(jnp.where(upd, base + lane, mi))
                return tuple(out) + (bx1, by1, bx2, by2, bar)

            m0 = jnp.full((L,), -3.0e38, jnp.float32)
            i0 = jnp.zeros((L,), jnp.int32)
            res = lax.fori_loop(0, ITERS, chunk,
                                (m0, i0) * U + (bx1, by1, bx2, by2, bar))
            mx, mi = res[0], res[1]
            for u in range(1, U):
                m2, i2 = res[2 * u], res[2 * u + 1]
                better = (m2 > mx) | ((m2 == mx) & (i2 < mi))
                mx = jnp.where(better, m2, mx)
                mi = jnp.where(better, i2, mi)
            return mx, mi

        far = jnp.full((L,), FAR, jnp.float32)
        m, mi = sweep(far, far, far, far, jnp.zeros((L,), jnp.float32))

        def step(k, carry):
            mx, mi = carry
            lane = lax.iota(jnp.int32, L)
            gm = jnp.max(mx)
            gmv = jnp.broadcast_to(gm, (L,))
            cand = jnp.where(mx == gmv, mi, jnp.int32(2**30))
            idx = jnp.min(cand)
            idxv = jnp.broadcast_to(idx, (L,))
            bx1 = plsc.load_gather(x1_v, [idxv])
            by1 = plsc.load_gather(y1_v, [idxv])
            bx2 = plsc.load_gather(x2_v, [idxv])
            by2 = plsc.load_gather(y2_v, [idxv])
            bar = (bx2 - bx1) * (by2 - by1)
            validv = jnp.where(gmv > CONF_T,
                               jnp.ones((L,), jnp.float32),
                               jnp.zeros((L,), jnp.float32))
            row = ((lane == 0).astype(jnp.float32) * gmv
                   + (lane == 1).astype(jnp.float32) * bx1
                   + (lane == 2).astype(jnp.float32) * by1
                   + (lane == 3).astype(jnp.float32) * bx2
                   + (lane == 4).astype(jnp.float32) * by2) * validv
            row_v[pl.ds(k * ROWW, ROWW)] = row
            plsc.store_scatter(s_v, [idxv], jnp.full((L,), NEG, jnp.float32),
                               mask=lane == 0)
            return sweep(bx1, by1, bx2, by2, bar)

        lax.fori_loop(0, TOP_K, step, (m, mi))
        pltpu.sync_copy(row_v, out_hbm.at[pl.ds(wid * TOP_K * ROWW, TOP_K * ROWW)])


def _nms(scores, boxes):
    mesh = plsc.VectorSubcoreMesh(core_axis_name="c", subcore_axis_name="s")
    return pl.kernel(
        _nms_body,
        out_type=jax.ShapeDtypeStruct((FG * TOP_K * ROWW,), jnp.float32),
        mesh=mesh,
        compiler_params=pltpu.CompilerParams(needs_layout_passes=False),
        scratch_types=[
            pltpu.VMEM((NP,), jnp.float32),
            pltpu.VMEM((NP,), jnp.float32),
            pltpu.VMEM((NP,), jnp.float32),
            pltpu.VMEM((NP,), jnp.float32),
            pltpu.VMEM((NP,), jnp.float32),
            pltpu.VMEM((TOP_K * ROWW,), jnp.float32),
        ],
    )(scores, boxes)


def kernel(locations, confidences, prior_boxes):
    conf_t = confidences[0].T
    loc_t = locations[0].T
    pri_t = prior_boxes.T
    scores, boxes = _prep(conf_t, loc_t, pri_t)
    rows = _nms(scores.reshape(-1), boxes.reshape(-1))
    fg = rows.reshape(FG, TOP_K, ROWW)[:, :, :5]
    bg = jnp.zeros((1, TOP_K, 5), jnp.float32)
    return jnp.concatenate([bg, fg], axis=0)[None]


# speedup vs baseline: 9.8448x; 1.3526x over previous
import jax
import jax.numpy as jnp
from jax import lax
from jax.experimental import pallas as pl
from jax.experimental.pallas import tpu as pltpu
from jax.experimental.pallas import tpu_sc as plsc

C = 21
FG = C - 1
N = 20000
TOP_K = 200
NMS_T = 0.45
CONF_T = 0.01
VAR0, VAR1 = 0.1, 0.2
NEG = -1e9
FAR = 1e30
L = 16
U = 4
NP = 20480
ITERS = NP // (U * L)
NC = 2
ROWW = 16


def _prep_body(conf_ref, loc_ref, pri_ref, scores_ref, boxes_ref):
    conf = conf_ref[...]
    m = jnp.max(conf, axis=0, keepdims=True)
    e = jnp.exp(conf - m)
    p = e / jnp.sum(e, axis=0, keepdims=True)
    fg = p[1:, :]
    spad = jnp.full((FG, NP - N), NEG, jnp.float32)
    scores_ref[...] = jnp.concatenate(
        [jnp.where(fg > CONF_T, fg, NEG), spad], axis=1)

    lxy = loc_ref[0:2, :]
    lwh = loc_ref[2:4, :]
    pxy = pri_ref[0:2, :]
    pwh = pri_ref[2:4, :]
    xy = pxy + lxy * VAR0 * pwh
    wh = pwh * jnp.exp(lwh * VAR1)
    tl = xy - wh / 2.0
    br = tl + wh
    bpad = jnp.full((4, NP - N), FAR, jnp.float32)
    boxes_ref[...] = jnp.concatenate(
        [jnp.concatenate([tl, br], axis=0), bpad], axis=1)


def _prep(conf_t, loc_t, pri_t):
    return pl.pallas_call(
        _prep_body,
        out_shape=[
            jax.ShapeDtypeStruct((FG, NP), jnp.float32),
            jax.ShapeDtypeStruct((4, NP), jnp.float32),
        ],
    )(conf_t, loc_t, pri_t)


def _nms_body(scores_hbm, boxes_hbm, out_hbm,
              s_v, idx_v, x1_v, y1_v, x2_v, y2_v, row_v):
    cid = lax.axis_index("c")
    sid = lax.axis_index("s")
    wid = sid * NC + cid

    @pl.when(wid < FG)
    def _work():
        pltpu.sync_copy(scores_hbm.at[pl.ds(wid * NP, NP)], s_v.at[pl.ds(0, NP)])
        pltpu.sync_copy(boxes_hbm.at[pl.ds(0 * NP, NP)], x1_v)
        pltpu.sync_copy(boxes_hbm.at[pl.ds(1 * NP, NP)], y1_v)
        pltpu.sync_copy(boxes_hbm.at[pl.ds(2 * NP, NP)], x2_v)
        pltpu.sync_copy(boxes_hbm.at[pl.ds(3 * NP, NP)], y2_v)

        def initidx(j, c):
            lane = lax.iota(jnp.int32, L)
            base = j * L
            idx_v[pl.ds(base, L)] = base + lane
            return c
        lax.fori_loop(0, NP // L, initidx, 0)

        def sweep(nr, bx1, by1, bx2, by2, bar):
            def chunk(base, carry):
                mx, mi, nm, bx1, by1, bx2, by2, bar = carry
                s = s_v[pl.ds(base, L)]
                idxs = idx_v[pl.ds(base, L)]
                x1 = plsc.load_gather(x1_v, [idxs])
                y1 = plsc.load_gather(y1_v, [idxs])
                x2 = plsc.load_gather(x2_v, [idxs])
                y2 = plsc.load_gather(y2_v, [idxs])
                ar = (x2 - x1) * (y2 - y1)
                ix1 = jnp.maximum(x1, bx1)
                iy1 = jnp.maximum(y1, by1)
                ix2 = jnp.minimum(x2, bx2)
                iy2 = jnp.minimum(y2, by2)
                iw = jnp.maximum(ix2 - ix1, 0.0)
                ih = jnp.maximum(iy2 - iy1, 0.0)
                inter = iw * ih
                denom = ((bar + ar) - inter) + 1e-9
                iou = inter / denom
                kill = iou > NMS_T
                s2 = jnp.where(kill, jnp.full((L,), NEG, jnp.float32), s)
                keep = s2 > 0.0
                incl = plsc.cumsum(keep.astype(jnp.int32))
                dest = nm + incl
                plsc.store_scatter(s_v, [dest], s2, mask=keep)
                plsc.store_scatter(idx_v, [dest], idxs, mask=keep)
                nm2 = nm + plsc.all_reduce_population_count(keep)
                mx2 = jnp.maximum(mx, s2)
                upd = mx2 > mx
                mi2 = jnp.where(upd, idxs, mi)
                return (mx2, mi2, nm2, bx1, by1, bx2, by2, bar)

            m0 = jnp.full((L,), -3.0e38, jnp.float32)
            i0 = jnp.zeros((L,), jnp.int32)
            nm0 = jnp.full((L,), -1, jnp.int32)
            res = plsc.parallel_loop(
                0, nr, L, unroll=U,
                carry=(m0, i0, nm0, bx1, by1, bx2, by2, bar))(chunk)
            mx, mi, nmf = res[0], res[1], res[2]
            lane = lax.iota(jnp.int32, L)
            negv = jnp.full((L,), NEG, jnp.float32)
            for u in range(U):
                plsc.store_scatter(s_v, [nmf + (1 + u * L) + lane], negv)
            n = jnp.max(nmf) + 1
            nr2 = ((n + (U * L - 1)) // (U * L)) * (U * L)
            return mx, mi, nr2

        far = jnp.full((L,), FAR, jnp.float32)
        m, mi, nr = sweep(NP, far, far, far, far,
                          jnp.zeros((L,), jnp.float32))

        def step(k, carry):
            mx, mi, nr = carry
            lane = lax.iota(jnp.int32, L)
            gm = jnp.max(mx)
            gmv = jnp.broadcast_to(gm, (L,))
            cand = jnp.where(mx == gmv, mi, jnp.int32(2**30))
            idx = jnp.min(cand)
            idxv = jnp.broadcast_to(idx, (L,))
            bx1 = plsc.load_gather(x1_v, [idxv])
            by1 = plsc.load_gather(y1_v, [idxv])
            bx2 = plsc.load_gather(x2_v, [idxv])
            by2 = plsc.load_gather(y2_v, [idxv])
            bar = (bx2 - bx1) * (by2 - by1)
            validv = jnp.where(gmv > CONF_T,
                               jnp.ones((L,), jnp.float32),
                               jnp.zeros((L,), jnp.float32))
            row = ((lane == 0).astype(jnp.float32) * gmv
                   + (lane == 1).astype(jnp.float32) * bx1
                   + (lane == 2).astype(jnp.float32) * by1
                   + (lane == 3).astype(jnp.float32) * bx2
                   + (lane == 4).astype(jnp.float32) * by2) * validv
            row_v[pl.ds(k * ROWW, ROWW)] = row
            return sweep(nr, bx1, by1, bx2, by2, bar)

        lax.fori_loop(0, TOP_K, step, (m, mi, nr))
        pltpu.sync_copy(row_v, out_hbm.at[pl.ds(wid * TOP_K * ROWW, TOP_K * ROWW)])


def _nms(scores, boxes):
    mesh = plsc.VectorSubcoreMesh(core_axis_name="c", subcore_axis_name="s")
    return pl.kernel(
        _nms_body,
        out_type=jax.ShapeDtypeStruct((FG * TOP_K * ROWW,), jnp.float32),
        mesh=mesh,
        compiler_params=pltpu.CompilerParams(needs_layout_passes=False),
        scratch_types=[
            pltpu.VMEM((NP + U * L,), jnp.float32),
            pltpu.VMEM((NP + U * L,), jnp.int32),
            pltpu.VMEM((NP,), jnp.float32),
            pltpu.VMEM((NP,), jnp.float32),
            pltpu.VMEM((NP,), jnp.float32),
            pltpu.VMEM((NP,), jnp.float32),
            pltpu.VMEM((TOP_K * ROWW,), jnp.float32),
        ],
    )(scores, boxes)


def kernel(locations, confidences, prior_boxes):
    conf_t = confidences[0].T
    loc_t = locations[0].T
    pri_t = prior_boxes.T
    scores, boxes = _prep(conf_t, loc_t, pri_t)
    rows = _nms(scores.reshape(-1), boxes.reshape(-1))
    fg = rows.reshape(FG, TOP_K, ROWW)[:, :, :5]
    bg = jnp.zeros((1, TOP_K, 5), jnp.float32)
    return jnp.concatenate([bg, fg], axis=0)[None]


# speedup vs baseline: 66.8318x; 6.7885x over previous
import jax
import jax.numpy as jnp
from jax import lax
from jax.experimental import pallas as pl
from jax.experimental.pallas import tpu as pltpu
from jax.experimental.pallas import tpu_sc as plsc

C = 21
FG = C - 1
N = 20000
TOP_K = 200
NMS_T = 0.45
CONF_T = 0.01
VAR0, VAR1 = 0.1, 0.2
NEG = -1e9
FAR = 1e30
L = 16
U = 4
W = U * L
NP = 20480
NC = 2
ROWW = 16
CAP = 1024
TGTLO = 256
LO_BITS = 0x3C23D70A
HI_BITS = 0x40000000


def _prep_body(conf_ref, loc_ref, pri_ref, scores_ref, boxes_ref):
    conf = conf_ref[...]
    m = jnp.max(conf, axis=0, keepdims=True)
    e = jnp.exp(conf - m)
    p = e / jnp.sum(e, axis=0, keepdims=True)
    fg = p[1:, :]
    spad = jnp.full((FG, NP - N), NEG, jnp.float32)
    scores_ref[...] = jnp.concatenate(
        [jnp.where(fg > CONF_T, fg, NEG), spad], axis=1)

    lxy = loc_ref[0:2, :]
    lwh = loc_ref[2:4, :]
    pxy = pri_ref[0:2, :]
    pwh = pri_ref[2:4, :]
    xy = pxy + lxy * VAR0 * pwh
    wh = pwh * jnp.exp(lwh * VAR1)
    tl = xy - wh / 2.0
    br = tl + wh
    bpad = jnp.full((4, NP - N), FAR, jnp.float32)
    boxes_ref[...] = jnp.concatenate(
        [jnp.concatenate([tl, br], axis=0), bpad], axis=1)


def _prep(conf_t, loc_t, pri_t):
    return pl.pallas_call(
        _prep_body,
        out_shape=[
            jax.ShapeDtypeStruct((FG, NP), jnp.float32),
            jax.ShapeDtypeStruct((4, NP), jnp.float32),
        ],
    )(conf_t, loc_t, pri_t)


def _nms_body(scores_hbm, boxes_hbm, out_hbm,
              s_v, idx_v, c_s, c_idx, x1_v, y1_v, x2_v, y2_v, row_v):
    cid = lax.axis_index("c")
    sid = lax.axis_index("s")
    wid = sid * NC + cid

    def sweep(sref, iref, nr, bx1, by1, bx2, by2, bar):
        def chunk(base, carry):
            mx, mi, nm, bx1, by1, bx2, by2, bar = carry
            s = sref[pl.ds(base, L)]
            idxs = iref[pl.ds(base, L)]
            x1 = plsc.load_gather(x1_v, [idxs])
            y1 = plsc.load_gather(y1_v, [idxs])
            x2 = plsc.load_gather(x2_v, [idxs])
            y2 = plsc.load_gather(y2_v, [idxs])
            ar = (x2 - x1) * (y2 - y1)
            ix1 = jnp.maximum(x1, bx1)
            iy1 = jnp.maximum(y1, by1)
            ix2 = jnp.minimum(x2, bx2)
            iy2 = jnp.minimum(y2, by2)
            iw = jnp.maximum(ix2 - ix1, 0.0)
            ih = jnp.maximum(iy2 - iy1, 0.0)
            inter = iw * ih
            denom = ((bar + ar) - inter) + 1e-9
            iou = inter / denom
            kill = iou > NMS_T
            s2 = jnp.where(kill, jnp.full((L,), NEG, jnp.float32), s)
            keep = s2 > 0.0
            incl = plsc.cumsum(keep.astype(jnp.int32))
            dest = nm + incl
            plsc.store_scatter(sref, [dest], s2, mask=keep)
            plsc.store_scatter(iref, [dest], idxs, mask=keep)
            nm2 = nm + plsc.all_reduce_population_count(keep)
            mx2 = jnp.maximum(mx, s2)
            upd = mx2 > mx
            mi2 = jnp.where(upd, idxs, mi)
            return (mx2, mi2, nm2, bx1, by1, bx2, by2, bar)

        m0 = jnp.full((L,), -3.0e38, jnp.float32)
        i0 = jnp.zeros((L,), jnp.int32)
        nm0 = jnp.full((L,), -1, jnp.int32)
        res = plsc.parallel_loop(
            0, nr, L, unroll=U,
            carry=(m0, i0, nm0, bx1, by1, bx2, by2, bar))(chunk)
        mx, mi, nmf = res[0], res[1], res[2]
        lane = lax.iota(jnp.int32, L)
        negv = jnp.full((L,), NEG, jnp.float32)
        zi = jnp.zeros((L,), jnp.int32)
        for u in range(U):
            plsc.store_scatter(sref, [nmf + (1 + u * L) + lane], negv)
            plsc.store_scatter(iref, [nmf + (1 + u * L) + lane], zi)
        n = jnp.max(nmf) + 1
        nr2 = ((n + (W - 1)) >> 6) << 6
        return mx, mi, n, nr2

    def count_above(nrr, tbits):
        tv = plsc.bitcast(jnp.broadcast_to(tbits, (L,)), jnp.float32)

        def chunk(base, carry):
            cnt, tv = carry
            s = s_v[pl.ds(base, L)]
            return (cnt + plsc.all_reduce_population_count(s > tv), tv)

        res = plsc.parallel_loop(
            0, nrr, L, unroll=U, carry=(jnp.zeros((L,), jnp.int32), tv))(chunk)
        return jnp.max(res[0])

    @pl.when(wid < FG)
    def _work():
        pltpu.sync_copy(scores_hbm.at[pl.ds(wid * NP, NP)],
                        s_v.at[pl.ds(0, NP)])
        pltpu.sync_copy(boxes_hbm.at[pl.ds(0 * NP, NP)], x1_v)
        pltpu.sync_copy(boxes_hbm.at[pl.ds(1 * NP, NP)], y1_v)
        pltpu.sync_copy(boxes_hbm.at[pl.ds(2 * NP, NP)], x2_v)
        pltpu.sync_copy(boxes_hbm.at[pl.ds(3 * NP, NP)], y2_v)

        def initidx(j, c):
            lane = lax.iota(jnp.int32, L)
            base = j * L
            idx_v[pl.ds(base, L)] = base + lane
            return c
        lax.fori_loop(0, NP // L, initidx, 0)

        far = jnp.full((L,), FAR, jnp.float32)
        zf = jnp.zeros((L,), jnp.float32)
        _, _, nrem0, nremr0 = sweep(s_v, idx_v, NP, far, far, far, far, zf)

        def refill(ops):
            mx, mi, ncr, nrem, nremr, k = ops
            del mx, mi, ncr
            def scond(st):
                lo, hi, cnt = st
                return (cnt > CAP) & (hi - lo > 1)

            def sbody(st):
                lo, hi, cnt = st
                mid = (lo + hi) >> 1
                c = count_above(nremr, mid)
                ok = c >= TGTLO
                return (jnp.where(ok, mid, lo), jnp.where(ok, hi, mid),
                        jnp.where(ok, c, cnt))

            lo, _, _ = lax.while_loop(
                scond, sbody, (jnp.int32(LO_BITS), jnp.int32(HI_BITS), nrem))
            tv = plsc.bitcast(jnp.broadcast_to(lo, (L,)), jnp.float32)

            def split(base, carry):
                cn, rn, tv = carry
                s = s_v[pl.ds(base, L)]
                idxs = idx_v[pl.ds(base, L)]
                alive = s > 0.0
                want = s > tv
                cdest = cn + plsc.cumsum(want.astype(jnp.int32))
                toc = want & (cdest < CAP)
                tor = alive & (~toc)
                rdest = rn + plsc.cumsum(tor.astype(jnp.int32))
                plsc.store_scatter(c_s, [cdest], s, mask=toc)
                plsc.store_scatter(c_idx, [cdest], idxs, mask=toc)
                plsc.store_scatter(s_v, [rdest], s, mask=tor)
                plsc.store_scatter(idx_v, [rdest], idxs, mask=tor)
                cn2 = cn + plsc.all_reduce_population_count(toc)
                rn2 = rn + plsc.all_reduce_population_count(tor)
                return (cn2, rn2, tv)

            nm1 = jnp.full((L,), -1, jnp.int32)
            cnf, rnf, _ = plsc.parallel_loop(
                0, nremr, L, unroll=U, carry=(nm1, nm1, tv))(split)
            lane = lax.iota(jnp.int32, L)
            negv = jnp.full((L,), NEG, jnp.float32)
            zi = jnp.zeros((L,), jnp.int32)
            for u in range(U):
                off = 1 + u * L
                plsc.store_scatter(c_s, [cnf + off + lane], negv)
                plsc.store_scatter(c_idx, [cnf + off + lane], zi)
                plsc.store_scatter(s_v, [rnf + off + lane], negv)
                plsc.store_scatter(idx_v, [rnf + off + lane], zi)
            nc = jnp.max(cnf) + 1
            ncr = ((nc + (W - 1)) >> 6) << 6
            nrem2 = jnp.max(rnf) + 1
            nremr2 = ((nrem2 + (W - 1)) >> 6) << 6

            def filt(j, ncr):
                bx1 = plsc.load_gather(row_v, [jnp.broadcast_to(j * ROWW + 1, (L,))])
                by1 = plsc.load_gather(row_v, [jnp.broadcast_to(j * ROWW + 2, (L,))])
                bx2 = plsc.load_gather(row_v, [jnp.broadcast_to(j * ROWW + 3, (L,))])
                by2 = plsc.load_gather(row_v, [jnp.broadcast_to(j * ROWW + 4, (L,))])
                bar = (bx2 - bx1) * (by2 - by1)

                def fchunk(base, carry):
                    bx1, by1, bx2, by2, bar = carry
                    s = c_s[pl.ds(base, L)]
                    idxs = c_idx[pl.ds(base, L)]
                    x1 = plsc.load_gather(x1_v, [idxs])
                    y1 = plsc.load_gather(y1_v, [idxs])
                    x2 = plsc.load_gather(x2_v, [idxs])
                    y2 = plsc.load_gather(y2_v, [idxs])
                    ar = (x2 - x1) * (y2 - y1)
                    ix1 = jnp.maximum(x1, bx1)
                    iy1 = jnp.maximum(y1, by1)
                    ix2 = jnp.minimum(x2, bx2)
                    iy2 = jnp.minimum(y2, by2)
                    iw = jnp.maximum(ix2 - ix1, 0.0)
                    ih = jnp.maximum(iy2 - iy1, 0.0)
                    inter = iw * ih
                    denom = ((bar + ar) - inter) + 1e-9
                    iou = inter / denom
                    s2 = jnp.where(iou > NMS_T,
                                   jnp.full((L,), NEG, jnp.float32), s)
                    c_s[pl.ds(base, L)] = s2
                    return carry

                plsc.parallel_loop(
                    0, ncr, L, unroll=U,
                    carry=(bx1, by1, bx2, by2, bar))(fchunk)
                return ncr

            ncr = lax.fori_loop(0, k, filt, ncr)

            farv = jnp.full((L,), FAR, jnp.float32)
            mx, mi, _, ncr2 = sweep(c_s, c_idx, ncr, farv, farv, farv, farv,
                                    jnp.zeros((L,), jnp.float32))
            return (mx, mi, ncr2, nrem2, nremr2)

        def no_refill(ops):
            mx, mi, ncr, nrem, nremr, k = ops
            del k
            return (mx, mi, ncr, nrem, nremr)

        def step(k, carry):
            mx, mi, ncr, nrem, nremr = carry
            gm0 = jnp.max(mx)
            need = (gm0 <= CONF_T) & (nrem > 0)
            mx, mi, ncr, nrem, nremr = lax.cond(
                need, refill, no_refill, (mx, mi, ncr, nrem, nremr, k))

            lane = lax.iota(jnp.int32, L)
            gm = jnp.max(mx)
            gmv = jnp.broadcast_to(gm, (L,))
            cand = jnp.where(mx == gmv, mi, jnp.int32(2**30))
            idx = jnp.min(cand)
            idxv = jnp.broadcast_to(idx, (L,))
            bx1 = plsc.load_gather(x1_v, [idxv])
            by1 = plsc.load_gather(y1_v, [idxv])
            bx2 = plsc.load_gather(x2_v, [idxv])
            by2 = plsc.load_gather(y2_v, [idxv])
            bar = (bx2 - bx1) * (by2 - by1)
            validv = jnp.where(gmv > CONF_T,
                               jnp.ones((L,), jnp.float32),
                               jnp.zeros((L,), jnp.float32))
            row = ((lane == 0).astype(jnp.float32) * gmv
                   + (lane == 1).astype(jnp.float32) * bx1
                   + (lane == 2).astype(jnp.float32) * by1
                   + (lane == 3).astype(jnp.float32) * bx2
                   + (lane == 4).astype(jnp.float32) * by2) * validv
            row_v[pl.ds(k * ROWW, ROWW)] = row
            mx2, mi2, _, ncr2 = sweep(c_s, c_idx, ncr, bx1, by1, bx2, by2, bar)
            return (mx2, mi2, ncr2, nrem, nremr)

        m0 = jnp.full((L,), -3.0e38, jnp.float32)
        i0 = jnp.zeros((L,), jnp.int32)
        lax.fori_loop(0, TOP_K, step, (m0, i0, jnp.int32(0), nrem0, nremr0))
        pltpu.sync_copy(row_v, out_hbm.at[pl.ds(wid * TOP_K * ROWW, TOP_K * ROWW)])


def _nms(scores, boxes):
    mesh = plsc.VectorSubcoreMesh(core_axis_name="c", subcore_axis_name="s")
    return pl.kernel(
        _nms_body,
        out_type=jax.ShapeDtypeStruct((FG * TOP_K * ROWW,), jnp.float32),
        mesh=mesh,
        compiler_params=pltpu.CompilerParams(needs_layout_passes=False),
        scratch_types=[
            pltpu.VMEM((NP + W,), jnp.float32),
            pltpu.VMEM((NP + W,), jnp.int32),
            pltpu.VMEM((CAP + W,), jnp.float32),
            pltpu.VMEM((CAP + W,), jnp.int32),
            pltpu.VMEM((NP,), jnp.float32),
            pltpu.VMEM((NP,), jnp.float32),
            pltpu.VMEM((NP,), jnp.float32),
            pltpu.VMEM((NP,), jnp.float32),
            pltpu.VMEM((TOP_K * ROWW,), jnp.float32),
        ],
    )(scores, boxes)


def kernel(locations, confidences, prior_boxes):
    conf_t = confidences[0].T
    loc_t = locations[0].T
    pri_t = prior_boxes.T
    scores, boxes = _prep(conf_t, loc_t, pri_t)
    rows = _nms(scores.reshape(-1), boxes.reshape(-1))
    fg = rows.reshape(FG, TOP_K, ROWW)[:, :, :5]
    bg = jnp.zeros((1, TOP_K, 5), jnp.float32)
    return jnp.concatenate([bg, fg], axis=0)[None]


# speedup vs baseline: 66.8832x; 1.0008x over previous
import jax
import jax.numpy as jnp
from jax import lax
from jax.experimental import pallas as pl
from jax.experimental.pallas import tpu as pltpu
from jax.experimental.pallas import tpu_sc as plsc

C = 21
FG = C - 1
N = 20000
TOP_K = 200
NMS_T = 0.45
CONF_T = 0.01
VAR0, VAR1 = 0.1, 0.2
NEG = -1e9
FAR = 1e30
L = 16
U = 4
W = U * L
NP = 20480
NC = 2
ROWW = 16
CAP = 1024
TGTLO = 512
LO_BITS = 0x3C23D70A
HI_BITS = 0x40000000


def _prep_body(conf_ref, loc_ref, pri_ref, scores_ref, boxes_ref):
    conf = conf_ref[...]
    m = jnp.max(conf, axis=0, keepdims=True)
    e = jnp.exp(conf - m)
    p = e / jnp.sum(e, axis=0, keepdims=True)
    fg = p[1:, :]
    spad = jnp.full((FG, NP - N), NEG, jnp.float32)
    scores_ref[...] = jnp.concatenate(
        [jnp.where(fg > CONF_T, fg, NEG), spad], axis=1)

    lxy = loc_ref[0:2, :]
    lwh = loc_ref[2:4, :]
    pxy = pri_ref[0:2, :]
    pwh = pri_ref[2:4, :]
    xy = pxy + lxy * VAR0 * pwh
    wh = pwh * jnp.exp(lwh * VAR1)
    tl = xy - wh / 2.0
    br = tl + wh
    bpad = jnp.full((4, NP - N), FAR, jnp.float32)
    boxes_ref[...] = jnp.concatenate(
        [jnp.concatenate([tl, br], axis=0), bpad], axis=1)


def _prep(conf_t, loc_t, pri_t):
    return pl.pallas_call(
        _prep_body,
        out_shape=[
            jax.ShapeDtypeStruct((FG, NP), jnp.float32),
            jax.ShapeDtypeStruct((4, NP), jnp.float32),
        ],
    )(conf_t, loc_t, pri_t)


def _nms_body(scores_hbm, boxes_hbm, out_hbm,
              s_v, idx_v, c_s, c_idx, x1_v, y1_v, x2_v, y2_v, row_v):
    cid = lax.axis_index("c")
    sid = lax.axis_index("s")
    wid = sid * NC + cid

    def sweep(sref, iref, nr, bx1, by1, bx2, by2, bar):
        def chunk(base, carry):
            mx, mi, nm, bx1, by1, bx2, by2, bar = carry
            s = sref[pl.ds(base, L)]
            idxs = iref[pl.ds(base, L)]
            x1 = plsc.load_gather(x1_v, [idxs])
            y1 = plsc.load_gather(y1_v, [idxs])
            x2 = plsc.load_gather(x2_v, [idxs])
            y2 = plsc.load_gather(y2_v, [idxs])
            ar = (x2 - x1) * (y2 - y1)
            ix1 = jnp.maximum(x1, bx1)
            iy1 = jnp.maximum(y1, by1)
            ix2 = jnp.minimum(x2, bx2)
            iy2 = jnp.minimum(y2, by2)
            iw = jnp.maximum(ix2 - ix1, 0.0)
            ih = jnp.maximum(iy2 - iy1, 0.0)
            inter = iw * ih
            denom = ((bar + ar) - inter) + 1e-9
            iou = inter / denom
            kill = iou > NMS_T
            s2 = jnp.where(kill, jnp.full((L,), NEG, jnp.float32), s)
            keep = s2 > 0.0
            incl = plsc.cumsum(keep.astype(jnp.int32))
            dest = nm + incl
            plsc.store_scatter(sref, [dest], s2, mask=keep)
            plsc.store_scatter(iref, [dest], idxs, mask=keep)
            nm2 = nm + plsc.all_reduce_population_count(keep)
            mx2 = jnp.maximum(mx, s2)
            upd = mx2 > mx
            mi2 = jnp.where(upd, idxs, mi)
            return (mx2, mi2, nm2, bx1, by1, bx2, by2, bar)

        m0 = jnp.full((L,), -3.0e38, jnp.float32)
        i0 = jnp.zeros((L,), jnp.int32)
        nm0 = jnp.full((L,), -1, jnp.int32)
        res = plsc.parallel_loop(
            0, nr, L, unroll=U,
            carry=(m0, i0, nm0, bx1, by1, bx2, by2, bar))(chunk)
        mx, mi, nmf = res[0], res[1], res[2]
        lane = lax.iota(jnp.int32, L)
        negv = jnp.full((L,), NEG, jnp.float32)
        for u in range(U):
            plsc.store_scatter(sref, [nmf + (1 + u * L) + lane], negv)
        n = jnp.max(nmf) + 1
        nr2 = ((n + (W - 1)) >> 6) << 6
        return mx, mi, n, nr2

    def count_above(nrr, tbits):
        tv = plsc.bitcast(jnp.broadcast_to(tbits, (L,)), jnp.float32)

        def chunk(base, carry):
            cnt, tv = carry
            s = s_v[pl.ds(base, L)]
            return (cnt + plsc.all_reduce_population_count(s > tv), tv)

        res = plsc.parallel_loop(
            0, nrr, L, unroll=U, carry=(jnp.zeros((L,), jnp.int32), tv))(chunk)
        return jnp.max(res[0])

    @pl.when(wid < FG)
    def _work():
        pltpu.sync_copy(scores_hbm.at[pl.ds(wid * NP, NP)],
                        s_v.at[pl.ds(0, NP)])
        pltpu.sync_copy(boxes_hbm.at[pl.ds(0 * NP, NP)], x1_v)
        pltpu.sync_copy(boxes_hbm.at[pl.ds(1 * NP, NP)], y1_v)
        pltpu.sync_copy(boxes_hbm.at[pl.ds(2 * NP, NP)], x2_v)
        pltpu.sync_copy(boxes_hbm.at[pl.ds(3 * NP, NP)], y2_v)

        def initidx(j, c):
            lane = lax.iota(jnp.int32, L)
            base = j * L
            idx_v[pl.ds(base, L)] = base + lane
            return c
        lax.fori_loop(0, NP // L, initidx, 0)

        far = jnp.full((L,), FAR, jnp.float32)
        zf = jnp.zeros((L,), jnp.float32)
        _, _, nrem0, nremr0 = sweep(s_v, idx_v, NP, far, far, far, far, zf)

        def refill(ops):
            mx, mi, ncr, nrem, nremr, k, gm0 = ops
            del mx, mi, ncr, gm0
            def scond(st):
                lo, hi, cnt = st
                return (cnt > CAP) & (hi - lo > 1)

            def sbody(st):
                lo, hi, cnt = st
                mid = (lo + hi) >> 1
                c = count_above(nremr, mid)
                ok = c >= TGTLO
                return (jnp.where(ok, mid, lo), jnp.where(ok, hi, mid),
                        jnp.where(ok, c, cnt))

            lo, _, _ = lax.while_loop(
                scond, sbody, (jnp.int32(LO_BITS), jnp.int32(HI_BITS), nrem))
            tv = plsc.bitcast(jnp.broadcast_to(lo, (L,)), jnp.float32)

            def split(base, carry):
                cn, rn, tv = carry
                s = s_v[pl.ds(base, L)]
                idxs = idx_v[pl.ds(base, L)]
                alive = s > 0.0
                want = s > tv
                cdest = cn + plsc.cumsum(want.astype(jnp.int32))
                toc = want & (cdest < CAP)
                tor = alive & (~toc)
                rdest = rn + plsc.cumsum(tor.astype(jnp.int32))
                plsc.store_scatter(c_s, [cdest], s, mask=toc)
                plsc.store_scatter(c_idx, [cdest], idxs, mask=toc)
                plsc.store_scatter(s_v, [rdest], s, mask=tor)
                plsc.store_scatter(idx_v, [rdest], idxs, mask=tor)
                cn2 = cn + plsc.all_reduce_population_count(toc)
                rn2 = rn + plsc.all_reduce_population_count(tor)
                return (cn2, rn2, tv)

            nm1 = jnp.full((L,), -1, jnp.int32)
            cnf, rnf, _ = plsc.parallel_loop(
                0, nremr, L, unroll=U, carry=(nm1, nm1, tv))(split)
            lane = lax.iota(jnp.int32, L)
            negv = jnp.full((L,), NEG, jnp.float32)
            zi = jnp.zeros((L,), jnp.int32)
            for u in range(U):
                off = 1 + u * L
                plsc.store_scatter(c_s, [cnf + off + lane], negv)
                plsc.store_scatter(c_idx, [cnf + off + lane], zi)
                plsc.store_scatter(s_v, [rnf + off + lane], negv)
                plsc.store_scatter(idx_v, [rnf + off + lane], zi)
            nc = jnp.max(cnf) + 1
            ncr = ((nc + (W - 1)) >> 6) << 6
            nrem2 = jnp.max(rnf) + 1
            nremr2 = ((nrem2 + (W - 1)) >> 6) << 6

            def filt(j, ncr):
                bx1 = plsc.load_gather(row_v, [jnp.broadcast_to(j * ROWW + 1, (L,))])
                by1 = plsc.load_gather(row_v, [jnp.broadcast_to(j * ROWW + 2, (L,))])
                bx2 = plsc.load_gather(row_v, [jnp.broadcast_to(j * ROWW + 3, (L,))])
                by2 = plsc.load_gather(row_v, [jnp.broadcast_to(j * ROWW + 4, (L,))])
                bar = (bx2 - bx1) * (by2 - by1)

                def fchunk(base, carry):
                    bx1, by1, bx2, by2, bar = carry
                    s = c_s[pl.ds(base, L)]
                    idxs = c_idx[pl.ds(base, L)]
                    x1 = plsc.load_gather(x1_v, [idxs])
                    y1 = plsc.load_gather(y1_v, [idxs])
                    x2 = plsc.load_gather(x2_v, [idxs])
                    y2 = plsc.load_gather(y2_v, [idxs])
                    ar = (x2 - x1) * (y2 - y1)
                    ix1 = jnp.maximum(x1, bx1)
                    iy1 = jnp.maximum(y1, by1)
                    ix2 = jnp.minimum(x2, bx2)
                    iy2 = jnp.minimum(y2, by2)
                    iw = jnp.maximum(ix2 - ix1, 0.0)
                    ih = jnp.maximum(iy2 - iy1, 0.0)
                    inter = iw * ih
                    denom = ((bar + ar) - inter) + 1e-9
                    iou = inter / denom
                    s2 = jnp.where(iou > NMS_T,
                                   jnp.full((L,), NEG, jnp.float32), s)
                    c_s[pl.ds(base, L)] = s2
                    return carry

                plsc.parallel_loop(
                    0, ncr, L, unroll=U,
                    carry=(bx1, by1, bx2, by2, bar))(fchunk)
                return ncr

            ncr = lax.fori_loop(0, k, filt, ncr)

            farv = jnp.full((L,), FAR, jnp.float32)
            mx, mi, _, ncr2 = sweep(c_s, c_idx, ncr, farv, farv, farv, farv,
                                    jnp.zeros((L,), jnp.float32))
            return (mx, mi, jnp.max(mx), ncr2, nrem2, nremr2)

        def no_refill(ops):
            mx, mi, ncr, nrem, nremr, k, gm0 = ops
            del k
            return (mx, mi, gm0, ncr, nrem, nremr)

        def step(k, carry):
            mx, mi, ncr, nrem, nremr = carry
            gm0 = jnp.max(mx)
            need = (gm0 <= CONF_T) & (nrem > 0)
            mx, mi, gm, ncr, nrem, nremr = lax.cond(
                need, refill, no_refill, (mx, mi, ncr, nrem, nremr, k, gm0))

            lane = lax.iota(jnp.int32, L)
            gmv = jnp.broadcast_to(gm, (L,))
            cand = jnp.where(mx == gmv, mi, jnp.int32(2**30))
            idx = jnp.min(cand)
            idxv = jnp.broadcast_to(idx, (L,))
            bx1 = plsc.load_gather(x1_v, [idxv])
            by1 = plsc.load_gather(y1_v, [idxv])
            bx2 = plsc.load_gather(x2_v, [idxv])
            by2 = plsc.load_gather(y2_v, [idxv])
            bar = (bx2 - bx1) * (by2 - by1)
            validv = jnp.where(gmv > CONF_T,
                               jnp.ones((L,), jnp.float32),
                               jnp.zeros((L,), jnp.float32))
            row = ((lane == 0).astype(jnp.float32) * gmv
                   + (lane == 1).astype(jnp.float32) * bx1
                   + (lane == 2).astype(jnp.float32) * by1
                   + (lane == 3).astype(jnp.float32) * bx2
                   + (lane == 4).astype(jnp.float32) * by2) * validv
            row_v[pl.ds(k * ROWW, ROWW)] = row
            mx2, mi2, _, ncr2 = sweep(c_s, c_idx, ncr, bx1, by1, bx2, by2, bar)
            return (mx2, mi2, ncr2, nrem, nremr)

        m0 = jnp.full((L,), -3.0e38, jnp.float32)
        i0 = jnp.zeros((L,), jnp.int32)
        lax.fori_loop(0, TOP_K, step, (m0, i0, jnp.int32(0), nrem0, nremr0))
        pltpu.sync_copy(row_v, out_hbm.at[pl.ds(wid * TOP_K * ROWW, TOP_K * ROWW)])


def _nms(scores, boxes):
    mesh = plsc.VectorSubcoreMesh(core_axis_name="c", subcore_axis_name="s")
    return pl.kernel(
        _nms_body,
        out_type=jax.ShapeDtypeStruct((FG * TOP_K * ROWW,), jnp.float32),
        mesh=mesh,
        compiler_params=pltpu.CompilerParams(needs_layout_passes=False),
        scratch_types=[
            pltpu.VMEM((NP + W,), jnp.float32),
            pltpu.VMEM((NP + W,), jnp.int32),
            pltpu.VMEM((CAP + W,), jnp.float32),
            pltpu.VMEM((CAP + W,), jnp.int32),
            pltpu.VMEM((NP,), jnp.float32),
            pltpu.VMEM((NP,), jnp.float32),
            pltpu.VMEM((NP,), jnp.float32),
            pltpu.VMEM((NP,), jnp.float32),
            pltpu.VMEM((TOP_K * ROWW,), jnp.float32),
        ],
    )(scores, boxes)


def kernel(locations, confidences, prior_boxes):
    conf_t = confidences[0].T
    loc_t = locations[0].T
    pri_t = prior_boxes.T
    scores, boxes = _prep(conf_t, loc_t, pri_t)
    rows = _nms(scores.reshape(-1), boxes.reshape(-1))
    fg = rows.reshape(FG, TOP_K, ROWW)[:, :, :5]
    bg = jnp.zeros((1, TOP_K, 5), jnp.float32)
    return jnp.concatenate([bg, fg], axis=0)[None]
